# baseline (speedup 1.0000x reference)
import functools
import math

import jax
import jax.numpy as jnp
from jax import lax
from jax.experimental import pallas as pl
from jax.experimental.pallas import tpu as pltpu
from jax.experimental.pallas import tpu_sc as plsc

_NC = 2
_NS = 16
_NW = _NC * _NS
_K = 80
_RB = 80


def _sc_spmm(X, idx2, vals3):
    N, D = X.shape
    n_blk, b_ch = idx2.shape[2], idx2.shape[3]
    n_rb = N // _RB
    assert b_ch % 2 == 1
    assert n_rb * _RB == N and D % 16 == 0 and _K % 16 == 0

    mesh = plsc.VectorSubcoreMesh(core_axis_name="c", subcore_axis_name="s")

    @functools.partial(
        pl.kernel,
        out_type=jax.ShapeDtypeStruct((_NC, N, D), jnp.float32),
        mesh=mesh,
        scratch_types=[
            pltpu.VMEM((2, b_ch, _K), jnp.int32),
            pltpu.VMEM((b_ch, _K), jnp.float32),
            [pltpu.VMEM((_K, D), jnp.float32)] * 2,
            pltpu.VMEM_SHARED((N, D), jnp.float32),
            [pltpu.SemaphoreType.DMA] * 2,
        ],
    )
    def spmm(x_hbm, idx_hbm, vals_hbm, out_hbm, blk, valb, bufs, acc, gsems):
        cid = lax.axis_index("c")
        sid = lax.axis_index("s")
        wid = sid * _NC + cid

        my_nb = (n_rb // _NS) + jnp.where(sid < (n_rb % _NS), 1, 0)
        zero = jnp.zeros((16,), jnp.float32)

        def zrow(r, carry):
            for j in range(D // 16):
                bufs[0][r, pl.ds(j * 16, 16)] = zero
            return carry

        lax.fori_loop(0, _RB, zrow, 0)

        def zblk(i, carry):
            off = pl.multiple_of((sid + i * _NS) * _RB, 8)
            pltpu.sync_copy(bufs[0], acc.at[pl.ds(off, _RB)])
            return carry

        lax.fori_loop(0, my_nb, zblk, 0)
        plsc.subcore_barrier()

        def start_gather(i, b):
            pltpu.async_copy(x_hbm.at[blk.at[1, i]], bufs[b], gsems[b])

        def process(j, b):
            pltpu.make_async_copy(x_hbm.at[blk.at[1, j]], bufs[b],
                                  gsems[b]).wait()
            buf = bufs[b]

            def grp(g, c2):
                v16 = valb[j, pl.ds(g * 16, 16)]
                for r2 in range(16):
                    v = v16[r2]
                    for j8 in range(D // 16):
                        sl = pl.ds(j8 * 16, 16)
                        buf[g * 16 + r2, sl] = buf[g * 16 + r2, sl] * v
                return c2

            lax.fori_loop(0, _K // 16, grp, 0)
            pltpu.sync_copy(buf, acc.at[blk.at[0, j]], add=True)

        def block(t, carry):
            pltpu.sync_copy(idx_hbm.at[wid, pl.ds(0, 2), t], blk)
            pltpu.sync_copy(vals_hbm.at[wid, t], valb)
            start_gather(0, 0)
            start_gather(1, 1)

            def pair(p, c2):
                for b in range(2):
                    j = 2 * p + b
                    process(j, b)
                    start_gather(j + 2, b)
                return c2

            lax.fori_loop(0, (b_ch - 3) // 2, pair, 0, unroll=2)
            process(b_ch - 3, (b_ch - 3) % 2)
            start_gather(b_ch - 1, (b_ch - 1) % 2)
            process(b_ch - 2, (b_ch - 2) % 2)
            process(b_ch - 1, (b_ch - 1) % 2)
            return carry

        lax.fori_loop(0, n_blk, block, 0)

        plsc.subcore_barrier()

        def oblk(i, carry):
            off = pl.multiple_of((sid + i * _NS) * _RB, 8)
            pltpu.sync_copy(acc.at[pl.ds(off, _RB)],
                            out_hbm.at[cid, pl.ds(off, _RB)])
            return carry

        lax.fori_loop(0, my_nb, oblk, 0)

    return spmm(X, idx2, vals3)


def _tc_combine(partials, h0, W):
    N, D = h0.shape
    BN = 400
    beta = math.log(0.5 / 4 + 1)

    def body(p_ref, h0_ref, w_ref, o_ref):
        support = 0.9 * (p_ref[0] + p_ref[1]) + 0.1 * h0_ref[...]
        o_ref[...] = beta * jnp.dot(support, w_ref[...],
                                    preferred_element_type=jnp.float32) \
            + (1.0 - beta) * support

    return pl.pallas_call(
        body,
        grid=(N // BN,),
        in_specs=[
            pl.BlockSpec((2, BN, D), lambda i: (0, i, 0)),
            pl.BlockSpec((BN, D), lambda i: (i, 0)),
            pl.BlockSpec((D, D), lambda i: (0, 0)),
        ],
        out_specs=pl.BlockSpec((BN, D), lambda i: (i, 0)),
        out_shape=jax.ShapeDtypeStruct((N, D), jnp.float32),
    )(partials, h0, W)


def kernel(X, h0, layer, edge_index, edge_vals, W):
    del layer
    E = edge_vals.shape[0]
    n_blk = 5
    b_ch = E // (_NW * _K * n_blk)
    assert n_blk * b_ch * _NW * _K == E
    shape4 = (_NW, n_blk, b_ch, _K)
    idx2 = jnp.stack([
        edge_index[0].reshape(shape4),
        edge_index[1].reshape(shape4),
    ], axis=1)
    partials = _sc_spmm(X, idx2, edge_vals.reshape(shape4))
    return _tc_combine(partials, h0, W)


# speedup vs baseline: 1.0089x; 1.0089x over previous
import functools
import math

import jax
import jax.numpy as jnp
from jax import lax
from jax.experimental import pallas as pl
from jax.experimental.pallas import tpu as pltpu
from jax.experimental.pallas import tpu_sc as plsc

_NC = 2
_NS = 16
_NW = _NC * _NS
_K = 80
_RB = 80


def _sc_spmm(X, idx2, vals3):
    N, D = X.shape
    n_blk, b_ch = idx2.shape[2], idx2.shape[3]
    n_rb = N // _RB
    assert b_ch % 2 == 1
    assert n_rb * _RB == N and D % 16 == 0 and _K % 16 == 0

    mesh = plsc.VectorSubcoreMesh(core_axis_name="c", subcore_axis_name="s")

    @functools.partial(
        pl.kernel,
        out_type=jax.ShapeDtypeStruct((_NC, N, D), jnp.float32),
        mesh=mesh,
        scratch_types=[
            pltpu.VMEM((2, b_ch, _K), jnp.int32),
            pltpu.VMEM((b_ch, _K), jnp.float32),
            [pltpu.VMEM((_K, D), jnp.float32)] * 2,
            pltpu.VMEM_SHARED((N, D), jnp.float32),
            [pltpu.SemaphoreType.DMA] * 2,
        ],
    )
    def spmm(x_hbm, idx_hbm, vals_hbm, out_hbm, blk, valb, bufs, acc, gsems):
        cid = lax.axis_index("c")
        sid = lax.axis_index("s")
        wid = sid * _NC + cid

        my_nb = (n_rb // _NS) + jnp.where(sid < (n_rb % _NS), 1, 0)
        zero = jnp.zeros((16,), jnp.float32)

        def zrow(r, carry):
            for j in range(D // 16):
                bufs[0][r, pl.ds(j * 16, 16)] = zero
            return carry

        lax.fori_loop(0, _RB, zrow, 0)

        def zblk(i, carry):
            off = pl.multiple_of((sid + i * _NS) * _RB, 8)
            pltpu.sync_copy(bufs[0], acc.at[pl.ds(off, _RB)])
            return carry

        lax.fori_loop(0, my_nb, zblk, 0)
        plsc.subcore_barrier()

        def start_gather(i, b):
            pltpu.async_copy(x_hbm.at[blk.at[1, i]], bufs[b], gsems[b])

        def process(j, b):
            pltpu.make_async_copy(x_hbm.at[blk.at[1, j]], bufs[b],
                                  gsems[b]).wait()
            buf = bufs[b]

            def grp(g, c2):
                v16 = valb[j, pl.ds(g * 16, 16)]
                for r2 in range(16):
                    v = v16[r2]
                    for j8 in range(D // 16):
                        sl = pl.ds(j8 * 16, 16)
                        buf[g * 16 + r2, sl] = buf[g * 16 + r2, sl] * v
                return c2

            lax.fori_loop(0, _K // 16, grp, 0)
            pltpu.sync_copy(buf, acc.at[blk.at[0, j]], add=True)

        def block(t, carry):
            pltpu.sync_copy(idx_hbm.at[wid, pl.ds(0, 2), t], blk)
            pltpu.sync_copy(vals_hbm.at[wid, t], valb)
            start_gather(0, 0)
            start_gather(1, 1)

            def pair(p, c2):
                for b in range(2):
                    j = 2 * p + b
                    process(j, b)
                    start_gather(j + 2, b)
                return c2

            lax.fori_loop(0, (b_ch - 3) // 2, pair, 0)
            process(b_ch - 3, (b_ch - 3) % 2)
            start_gather(b_ch - 1, (b_ch - 1) % 2)
            process(b_ch - 2, (b_ch - 2) % 2)
            process(b_ch - 1, (b_ch - 1) % 2)
            return carry

        lax.fori_loop(0, n_blk, block, 0)

        plsc.subcore_barrier()

        def oblk(i, carry):
            off = pl.multiple_of((sid + i * _NS) * _RB, 8)
            pltpu.sync_copy(acc.at[pl.ds(off, _RB)],
                            out_hbm.at[cid, pl.ds(off, _RB)])
            return carry

        lax.fori_loop(0, my_nb, oblk, 0)

    return spmm(X, idx2, vals3)


def _tc_combine(partials, h0, W):
    N, D = h0.shape
    BN = 400
    beta = math.log(0.5 / 4 + 1)

    def body(p_ref, h0_ref, w_ref, o_ref):
        support = 0.9 * (p_ref[0] + p_ref[1]) + 0.1 * h0_ref[...]
        o_ref[...] = beta * jnp.dot(support, w_ref[...],
                                    preferred_element_type=jnp.float32) \
            + (1.0 - beta) * support

    return pl.pallas_call(
        body,
        grid=(N // BN,),
        in_specs=[
            pl.BlockSpec((2, BN, D), lambda i: (0, i, 0)),
            pl.BlockSpec((BN, D), lambda i: (i, 0)),
            pl.BlockSpec((D, D), lambda i: (0, 0)),
        ],
        out_specs=pl.BlockSpec((BN, D), lambda i: (i, 0)),
        out_shape=jax.ShapeDtypeStruct((N, D), jnp.float32),
    )(partials, h0, W)


def kernel(X, h0, layer, edge_index, edge_vals, W):
    del layer
    E = edge_vals.shape[0]
    n_blk = 5
    b_ch = E // (_NW * _K * n_blk)
    assert n_blk * b_ch * _NW * _K == E
    shape4 = (_NW, n_blk, b_ch, _K)
    idx2 = jnp.stack([
        edge_index[0].reshape(shape4),
        edge_index[1].reshape(shape4),
    ], axis=1)
    partials = _sc_spmm(X, idx2, edge_vals.reshape(shape4))
    return _tc_combine(partials, h0, W)


# speedup vs baseline: 1.0658x; 1.0565x over previous
import functools
import math

import jax
import jax.numpy as jnp
from jax import lax
from jax.experimental import pallas as pl
from jax.experimental.pallas import tpu as pltpu
from jax.experimental.pallas import tpu_sc as plsc

_NC = 2
_NS = 16
_NW = _NC * _NS
_K = 80
_RB = 80


def _sc_spmm(X, idx2, vals3):
    N, D = X.shape
    n_blk, b_ch = idx2.shape[2], idx2.shape[3]
    n_rb = N // _RB
    assert b_ch % 2 == 1
    assert n_rb * _RB == N and D % 16 == 0 and _K % 16 == 0

    mesh = plsc.VectorSubcoreMesh(core_axis_name="c", subcore_axis_name="s")

    @functools.partial(
        pl.kernel,
        out_type=jax.ShapeDtypeStruct((_NC, N, D), jnp.float32),
        mesh=mesh,
        scratch_types=[
            pltpu.VMEM((2, b_ch, _K), jnp.int32),
            pltpu.VMEM((b_ch, _K), jnp.float32),
            [pltpu.VMEM((_K, D), jnp.float32)] * 2,
            pltpu.VMEM_SHARED((N, D), jnp.float32),
            [pltpu.SemaphoreType.DMA] * 2,
        ],
    )
    def spmm(x_hbm, idx_hbm, vals_hbm, out_hbm, blk, valb, bufs, acc, gsems):
        cid = lax.axis_index("c")
        sid = lax.axis_index("s")
        wid = sid * _NC + cid

        my_nb = (n_rb // _NS) + jnp.where(sid < (n_rb % _NS), 1, 0)
        zero = jnp.zeros((16,), jnp.float32)

        def zrow(r, carry):
            for j in range(D // 16):
                bufs[0][r, pl.ds(j * 16, 16)] = zero
            return carry

        lax.fori_loop(0, _RB, zrow, 0)

        def zblk(i, carry):
            off = pl.multiple_of((sid + i * _NS) * _RB, 8)
            pltpu.sync_copy(bufs[0], acc.at[pl.ds(off, _RB)])
            return carry

        lax.fori_loop(0, my_nb, zblk, 0)
        plsc.subcore_barrier()

        def start_gather(i, b):
            pltpu.async_copy(x_hbm.at[blk.at[1, i]], bufs[b], gsems[b])

        def process(j, b):
            pltpu.make_async_copy(x_hbm.at[blk.at[1, j]], bufs[b],
                                  gsems[b]).wait()
            buf = bufs[b]

            def grp(g, c2):
                v16 = valb[j, pl.ds(g * 16, 16)]
                for r2 in range(16):
                    v = v16[r2]
                    for j8 in range(D // 16):
                        sl = pl.ds(j8 * 16, 16)
                        buf[g * 16 + r2, sl] = buf[g * 16 + r2, sl] * v
                return c2

            lax.fori_loop(0, _K // 16, grp, 0)
            pltpu.sync_copy(buf, acc.at[blk.at[0, j]], add=True)

        def block(t, carry):
            pltpu.sync_copy(idx_hbm.at[wid, pl.ds(0, 2), t], blk)
            pltpu.sync_copy(vals_hbm.at[wid, t], valb)
            start_gather(0, 0)
            start_gather(1, 1)

            def pair(p, c2):
                for b in range(2):
                    j = 2 * p + b
                    process(j, b)
                    nxt = j + 2

                    @pl.when(nxt < b_ch)
                    def _():
                        start_gather(nxt, b)
                return c2

            lax.fori_loop(0, b_ch // 2, pair, 0)
            process(b_ch - 1, (b_ch - 1) % 2)
            return carry

        lax.fori_loop(0, n_blk, block, 0)

        plsc.subcore_barrier()

        def oblk(i, carry):
            off = pl.multiple_of((sid + i * _NS) * _RB, 8)
            pltpu.sync_copy(acc.at[pl.ds(off, _RB)],
                            out_hbm.at[cid, pl.ds(off, _RB)])
            return carry

        lax.fori_loop(0, my_nb, oblk, 0)

    return spmm(X, idx2, vals3)


def _tc_combine(partials, h0, W):
    N, D = h0.shape
    BN = 2000
    beta = math.log(0.5 / 4 + 1)

    def body(p_ref, h0_ref, w_ref, o_ref):
        support = 0.9 * (p_ref[0] + p_ref[1]) + 0.1 * h0_ref[...]
        o_ref[...] = beta * jnp.dot(support, w_ref[...],
                                    preferred_element_type=jnp.float32) \
            + (1.0 - beta) * support

    return pl.pallas_call(
        body,
        grid=(N // BN,),
        in_specs=[
            pl.BlockSpec((2, BN, D), lambda i: (0, i, 0)),
            pl.BlockSpec((BN, D), lambda i: (i, 0)),
            pl.BlockSpec((D, D), lambda i: (0, 0)),
        ],
        out_specs=pl.BlockSpec((BN, D), lambda i: (i, 0)),
        out_shape=jax.ShapeDtypeStruct((N, D), jnp.float32),
    )(partials, h0, W)


def kernel(X, h0, layer, edge_index, edge_vals, W):
    del layer
    E = edge_vals.shape[0]
    n_blk = 5
    b_ch = E // (_NW * _K * n_blk)
    assert n_blk * b_ch * _NW * _K == E
    shape4 = (_NW, n_blk, b_ch, _K)
    idx2 = jnp.stack([
        edge_index[0].reshape(shape4),
        edge_index[1].reshape(shape4),
    ], axis=1)
    partials = _sc_spmm(X, idx2, edge_vals.reshape(shape4))
    return _tc_combine(partials, h0, W)


# speedup vs baseline: 1.1564x; 1.0850x over previous
import functools
import math

import jax
import jax.numpy as jnp
from jax import lax
from jax.experimental import pallas as pl
from jax.experimental.pallas import tpu as pltpu
from jax.experimental.pallas import tpu_sc as plsc

_NC = 2
_NS = 16
_NW = _NC * _NS
_K = 80
_RB = 80


def _sc_spmm(X, idx2, vals3):
    N, D = X.shape
    n_blk, b_ch = idx2.shape[2], idx2.shape[3]
    assert idx2.shape[0] == 2 and idx2.shape[1] == _NW
    n_rb = N // _RB
    assert b_ch % 2 == 1
    assert n_rb * _RB == N and D % 16 == 0 and _K % 16 == 0

    mesh = plsc.VectorSubcoreMesh(core_axis_name="c", subcore_axis_name="s")

    @functools.partial(
        pl.kernel,
        out_type=jax.ShapeDtypeStruct((_NC, N, D), jnp.float32),
        mesh=mesh,
        scratch_types=[
            pltpu.VMEM((2, b_ch, _K), jnp.int32),
            pltpu.VMEM((b_ch, _K), jnp.float32),
            [pltpu.VMEM((_K, D), jnp.float32)] * 2,
            pltpu.VMEM_SHARED((N, D), jnp.float32),
            [pltpu.SemaphoreType.DMA] * 2,
        ],
    )
    def spmm(x_hbm, idx_hbm, vals_hbm, out_hbm, blk, valb, bufs, acc, gsems):
        cid = lax.axis_index("c")
        sid = lax.axis_index("s")
        wid = sid * _NC + cid

        my_nb = (n_rb // _NS) + jnp.where(sid < (n_rb % _NS), 1, 0)
        zero = jnp.zeros((16,), jnp.float32)

        def zrow(r, carry):
            for j in range(D // 16):
                bufs[0][r, pl.ds(j * 16, 16)] = zero
            return carry

        lax.fori_loop(0, _RB, zrow, 0)

        def zblk(i, carry):
            off = pl.multiple_of((sid + i * _NS) * _RB, 8)
            pltpu.sync_copy(bufs[0], acc.at[pl.ds(off, _RB)])
            return carry

        lax.fori_loop(0, my_nb, zblk, 0)
        plsc.subcore_barrier()

        def start_gather(i, b):
            pltpu.async_copy(x_hbm.at[blk.at[1, i]], bufs[b], gsems[b])

        def process(j, b):
            pltpu.make_async_copy(x_hbm.at[blk.at[1, j]], bufs[b],
                                  gsems[b]).wait()
            buf = bufs[b]

            def grp(g, c2):
                v16 = valb[j, pl.ds(g * 16, 16)]
                for r2 in range(16):
                    v = v16[r2]
                    for j8 in range(D // 16):
                        sl = pl.ds(j8 * 16, 16)
                        buf[g * 16 + r2, sl] = buf[g * 16 + r2, sl] * v
                return c2

            lax.fori_loop(0, _K // 16, grp, 0)
            pltpu.sync_copy(buf, acc.at[blk.at[0, j]], add=True)

        def block(t, carry):
            pltpu.sync_copy(idx_hbm.at[pl.ds(0, 2), wid, t], blk)
            pltpu.sync_copy(vals_hbm.at[wid, t], valb)
            start_gather(0, 0)
            start_gather(1, 1)

            def pair(p, c2):
                for b in range(2):
                    j = 2 * p + b
                    process(j, b)
                    nxt = j + 2

                    @pl.when(nxt < b_ch)
                    def _():
                        start_gather(nxt, b)
                return c2

            lax.fori_loop(0, b_ch // 2, pair, 0)
            process(b_ch - 1, (b_ch - 1) % 2)
            return carry

        lax.fori_loop(0, n_blk, block, 0)

        plsc.subcore_barrier()

        def oblk(i, carry):
            off = pl.multiple_of((sid + i * _NS) * _RB, 8)
            pltpu.sync_copy(acc.at[pl.ds(off, _RB)],
                            out_hbm.at[cid, pl.ds(off, _RB)])
            return carry

        lax.fori_loop(0, my_nb, oblk, 0)

    return spmm(X, idx2, vals3)


def _tc_combine(partials, h0, W):
    N, D = h0.shape
    BN = 2000
    beta = math.log(0.5 / 4 + 1)

    def body(p_ref, h0_ref, w_ref, o_ref):
        support = 0.9 * (p_ref[0] + p_ref[1]) + 0.1 * h0_ref[...]
        o_ref[...] = beta * jnp.dot(support, w_ref[...],
                                    preferred_element_type=jnp.float32) \
            + (1.0 - beta) * support

    return pl.pallas_call(
        body,
        grid=(N // BN,),
        in_specs=[
            pl.BlockSpec((2, BN, D), lambda i: (0, i, 0)),
            pl.BlockSpec((BN, D), lambda i: (i, 0)),
            pl.BlockSpec((D, D), lambda i: (0, 0)),
        ],
        out_specs=pl.BlockSpec((BN, D), lambda i: (i, 0)),
        out_shape=jax.ShapeDtypeStruct((N, D), jnp.float32),
    )(partials, h0, W)


def kernel(X, h0, layer, edge_index, edge_vals, W):
    del layer
    E = edge_vals.shape[0]
    n_blk = 5
    b_ch = E // (_NW * _K * n_blk)
    assert n_blk * b_ch * _NW * _K == E
    shape4 = (_NW, n_blk, b_ch, _K)
    idx2 = edge_index.reshape((2,) + shape4)
    partials = _sc_spmm(X, idx2, edge_vals.reshape(shape4))
    return _tc_combine(partials, h0, W)
